# Initial kernel scaffold; baseline (speedup 1.0000x reference)
import jax
import jax.numpy as jnp
from jax.experimental import pallas as pl


def kernel(feats, edge_index, edge_attr, W_rel, b_rel, W_edge, b_edge, W_res, b_res, gamma, beta):
    raise NotImplementedError("write your pallas kernel here")


# speedup vs baseline: 2.9250x; 2.9250x over previous
import jax
import jax.numpy as jnp
from jax import lax
from jax.experimental import pallas as pl
from jax.experimental.pallas import tpu as pltpu
from jax.experimental.pallas import tpu_sc as plsc

N_NODES = 10000
N_PAD = 10112
D_IN = 128
D_OUT = 128
D_EDGE = 16
D_DEG = 8
D_HALF = 64
N_EDGES = 320000
C = 128
NCHUNKS = N_EDGES // C
NC = 2
NS = 16
NW = NC * NS
CH_PER_W = -(-NCHUNKS // NW)
RPT = N_PAD // NS
ZCHUNKS = (128, 128, 128, 128, 120)


def _sc_body(featsS_hbm, src_hbm, dst_hbm, attr_hbm, ones_hbm, zeros_hbm,
             out_node, out_attr, out_deg,
             sidx, didx, rows, attrv, ones_v,
             acc_node, acc_attr, acc_deg, sem):
    cid = lax.axis_index("c")
    sid = lax.axis_index("s")
    wid = sid * NC + cid
    zero16 = jnp.zeros((16,), jnp.float32)

    def _fill_row(r, carry):
        for j in range(D_HALF // 16):
            rows[r, pl.ds(j * 16, 16)] = zero16
        attrv[r, :] = zero16
        return carry
    lax.fori_loop(0, C, _fill_row, 0)
    pltpu.sync_copy(ones_hbm, ones_v)

    r0 = sid * RPT
    zoff = 0
    for zc in ZCHUNKS:
        pltpu.sync_copy(rows.at[pl.ds(0, zc)], acc_node.at[pl.ds(r0 + zoff, zc)])
        pltpu.sync_copy(attrv.at[pl.ds(0, zc)], acc_attr.at[pl.ds(r0 + zoff, zc)])
        pltpu.sync_copy(zeros_hbm.at[pl.ds(0, zc)], acc_deg.at[pl.ds(r0 + zoff, zc)])
        zoff += zc
    plsc.subcore_barrier()

    off16 = jax.lax.broadcast(cid * N_NODES, (16,)).astype(jnp.int32)
    nch_n = NCHUNKS // NS + jnp.where(sid < NCHUNKS % NS, 1, 0)

    def _chunk_node(i, carry):
        base = (sid + NS * i) * C
        pltpu.sync_copy(src_hbm.at[pl.ds(base, C)], sidx)
        pltpu.sync_copy(dst_hbm.at[pl.ds(base, C)], didx)
        for j in range(C // 16):
            sidx[pl.ds(j * 16, 16)] = sidx[pl.ds(j * 16, 16)] + off16
        pltpu.async_copy(featsS_hbm.at[sidx], rows, sem).wait()
        pltpu.sync_copy(rows, acc_node.at[didx], add=True)
        return carry
    lax.fori_loop(0, nch_n, _chunk_node, 0)

    nch_a = NCHUNKS // NW + jnp.where(wid < NCHUNKS % NW, 1, 0)

    def _chunk_attr(i, carry):
        base = (wid + NW * i) * C
        pltpu.sync_copy(dst_hbm.at[pl.ds(base, C)], didx)
        pltpu.sync_copy(attr_hbm.at[pl.ds(base, C)], attrv)
        pltpu.sync_copy(attrv, acc_attr.at[didx], add=True)
        pltpu.sync_copy(ones_v, acc_deg.at[didx], add=True)
        return carry
    lax.fori_loop(0, nch_a, _chunk_attr, 0)
    plsc.subcore_barrier()

    pltpu.sync_copy(acc_node.at[pl.ds(r0, RPT)], out_node.at[cid, pl.ds(r0, RPT)])
    pltpu.sync_copy(acc_attr.at[pl.ds(r0, RPT)], out_attr.at[cid, pl.ds(r0, RPT)])
    pltpu.sync_copy(acc_deg.at[pl.ds(r0, RPT)], out_deg.at[cid, pl.ds(r0, RPT)])


def _sc_aggregate(featsS, src, dst, edge_attr, ones_d, zeros_d):
    mesh = plsc.VectorSubcoreMesh(core_axis_name="c", subcore_axis_name="s")
    kfn = pl.kernel(
        _sc_body,
        mesh=mesh,
        compiler_params=pltpu.CompilerParams(use_tc_tiling_on_sc=False),
        out_type=[
            jax.ShapeDtypeStruct((NC, N_PAD, D_HALF), jnp.float32),
            jax.ShapeDtypeStruct((NC, N_PAD, D_EDGE), jnp.float32),
            jax.ShapeDtypeStruct((NC, N_PAD, D_DEG), jnp.float32),
        ],
        scratch_types=[
            pltpu.VMEM((C,), jnp.int32),
            pltpu.VMEM((C,), jnp.int32),
            pltpu.VMEM((C, D_HALF), jnp.float32),
            pltpu.VMEM((C, D_EDGE), jnp.float32),
            pltpu.VMEM((C, D_DEG), jnp.float32),
            pltpu.VMEM_SHARED((N_PAD, D_HALF), jnp.float32),
            pltpu.VMEM_SHARED((N_PAD, D_EDGE), jnp.float32),
            pltpu.VMEM_SHARED((N_PAD, D_DEG), jnp.float32),
            pltpu.SemaphoreType.DMA,
        ],
    )
    return kfn(featsS, src, dst, edge_attr, ones_d, zeros_d)


def _combine_body(np_ref, ap_ref, dp_ref, feats_ref,
                  wrelt_ref, wedget_ref, wrest_ref,
                  bcomb_ref, bres_ref, gamma_ref, beta_ref, out_ref):
    aggf = jnp.concatenate(
        [np_ref[0, :N_NODES, :], np_ref[1, :N_NODES, :]], axis=1)
    segattr = ap_ref[0, :N_NODES, :] + ap_ref[1, :N_NODES, :]
    deg = dp_ref[0, :N_NODES, 0:1] + dp_ref[1, :N_NODES, 0:1]
    agg = (jnp.dot(aggf, wrelt_ref[...], preferred_element_type=jnp.float32)
           + jnp.dot(segattr, wedget_ref[...], preferred_element_type=jnp.float32)
           + deg * bcomb_ref[...])
    new = jnp.maximum(agg, 0.0)
    res = jnp.maximum(
        jnp.dot(feats_ref[...], wrest_ref[...], preferred_element_type=jnp.float32)
        + bres_ref[...], 0.0)
    new = new + res
    mean = jnp.mean(new, axis=0, keepdims=True)
    var = jnp.mean((new - mean) ** 2, axis=0, keepdims=True)
    out_ref[...] = (new - mean) * lax.rsqrt(var + 1e-5) * gamma_ref[...] + beta_ref[...]


def _combine(node_p, attr_p, deg_p, feats, wrelt, wedget, wrest,
             bcomb, bres, gamma, beta):
    return pl.pallas_call(
        _combine_body,
        out_shape=jax.ShapeDtypeStruct((N_NODES, D_OUT), jnp.float32),
    )(node_p, attr_p, deg_p, feats, wrelt, wedget, wrest, bcomb, bres, gamma, beta)


def kernel(feats, edge_index, edge_attr, W_rel, b_rel, W_edge, b_edge,
           W_res, b_res, gamma, beta):
    src = edge_index[0]
    dst = edge_index[1]
    featsS = jnp.concatenate([feats[:, :D_HALF], feats[:, D_HALF:]], axis=0)
    node_p, attr_p, deg_p = _sc_aggregate(
        featsS, src, dst, edge_attr,
        jnp.ones((C, D_DEG), jnp.float32), jnp.zeros((C, D_DEG), jnp.float32))
    return _combine(
        node_p, attr_p, deg_p, feats,
        W_rel.T, W_edge.T, W_res.T,
        (b_rel + b_edge).reshape(1, D_OUT), b_res.reshape(1, D_OUT),
        gamma.reshape(1, D_OUT), beta.reshape(1, D_OUT))


# speedup vs baseline: 4.6514x; 1.5902x over previous
import jax
import jax.numpy as jnp
from jax import lax
from jax.experimental import pallas as pl
from jax.experimental.pallas import tpu as pltpu
from jax.experimental.pallas import tpu_sc as plsc

N_NODES = 10000
N_PAD = 10112
D_IN = 128
D_OUT = 128
D_EDGE = 16
D_DEG = 8
D_HALF = 64
N_EDGES = 320000
C = 128
NCHUNKS = N_EDGES // C
NC = 2
NS = 16
NW = NC * NS
NSLOTS = 158
RPT = N_PAD // NS
ZCHUNKS = (128, 128, 128, 128, 120)


def _sc_body(featsS_hbm, src_hbm, dst_hbm, attr_hbm, ones_hbm, zeros_hbm,
             out_node, out_attr, out_deg,
             sidx0, didx0, rows0, attrv0, sidx1, didx1, rows1, attrv1, ones_v,
             acc_node, acc_attr, acc_deg,
             sem_s0, sem_d0, sem_a0, sem_g0, sem_s1, sem_d1, sem_a1, sem_g1):
    cid = lax.axis_index("c")
    sid = lax.axis_index("s")
    zero16 = jnp.zeros((16,), jnp.float32)
    off16 = lax.broadcast(cid * N_NODES, (16,)).astype(jnp.int32)
    dump16 = jnp.full((16,), N_NODES, jnp.int32)

    def _fill_row(r, carry):
        for j in range(D_HALF // 16):
            rows0[r, pl.ds(j * 16, 16)] = zero16
        attrv0[r, :] = zero16
        return carry
    lax.fori_loop(0, C, _fill_row, 0)
    pltpu.sync_copy(ones_hbm, ones_v)

    r0 = sid * RPT
    zoff = 0
    for zc in ZCHUNKS:
        pltpu.sync_copy(rows0.at[pl.ds(0, zc)], acc_node.at[pl.ds(r0 + zoff, zc)])
        pltpu.sync_copy(attrv0.at[pl.ds(0, zc)], acc_attr.at[pl.ds(r0 + zoff, zc)])
        pltpu.sync_copy(zeros_hbm.at[pl.ds(0, zc)], acc_deg.at[pl.ds(r0 + zoff, zc)])
        zoff += zc
    plsc.subcore_barrier()

    def _base(slot):
        ch = sid + NS * slot
        return jnp.minimum(ch, NCHUNKS - 1) * C

    def _valid16(slot):
        v = (sid + NS * slot < NCHUNKS).astype(jnp.int32)
        return lax.broadcast(v, (16,))

    def _fix(sidx, didx, v16):
        for j in range(C // 16):
            sl = pl.ds(j * 16, 16)
            sidx[sl] = sidx[sl] + off16
            didx[sl] = didx[sl] * v16 + dump16 * (1 - v16)

    def _start_loads(slot, sidx, didx, attrv, ss, sd, sa):
        b = _base(slot)
        pltpu.async_copy(src_hbm.at[pl.ds(b, C)], sidx, ss)
        pltpu.async_copy(dst_hbm.at[pl.ds(b, C)], didx, sd)
        pltpu.async_copy(attr_hbm.at[pl.ds(b, C)], attrv, sa)

    def _wait_loads(sidx, didx, attrv, ss, sd, sa):
        pltpu.make_async_copy(src_hbm.at[pl.ds(0, C)], sidx, ss).wait()
        pltpu.make_async_copy(dst_hbm.at[pl.ds(0, C)], didx, sd).wait()
        pltpu.make_async_copy(attr_hbm.at[pl.ds(0, C)], attrv, sa).wait()

    def _scatter(rows, attrv, didx):
        pltpu.sync_copy(rows, acc_node.at[didx], add=True)
        pltpu.sync_copy(attrv, acc_attr.at[didx], add=True)
        pltpu.sync_copy(ones_v, acc_deg.at[didx], add=True)

    b0 = _base(0)
    pltpu.sync_copy(src_hbm.at[pl.ds(b0, C)], sidx0)
    pltpu.sync_copy(dst_hbm.at[pl.ds(b0, C)], didx0)
    pltpu.sync_copy(attr_hbm.at[pl.ds(b0, C)], attrv0)
    _fix(sidx0, didx0, _valid16(0))
    pltpu.async_copy(featsS_hbm.at[sidx0], rows0, sem_g0)
    _start_loads(1, sidx1, didx1, attrv1, sem_s1, sem_d1, sem_a1)

    def _pair(p, carry):
        pltpu.make_async_copy(featsS_hbm.at[sidx0], rows0, sem_g0).wait()
        _wait_loads(sidx1, didx1, attrv1, sem_s1, sem_d1, sem_a1)
        _fix(sidx1, didx1, _valid16(2 * p + 1))
        gb = pltpu.async_copy(featsS_hbm.at[sidx1], rows1, sem_g1)
        _scatter(rows0, attrv0, didx0)
        _start_loads(2 * p + 2, sidx0, didx0, attrv0, sem_s0, sem_d0, sem_a0)
        gb.wait()
        _scatter(rows1, attrv1, didx1)
        _start_loads(2 * p + 3, sidx1, didx1, attrv1, sem_s1, sem_d1, sem_a1)
        _wait_loads(sidx0, didx0, attrv0, sem_s0, sem_d0, sem_a0)
        _fix(sidx0, didx0, _valid16(2 * p + 2))
        pltpu.async_copy(featsS_hbm.at[sidx0], rows0, sem_g0)
        return carry
    lax.fori_loop(0, NSLOTS // 2, _pair, 0)

    pltpu.make_async_copy(featsS_hbm.at[sidx0], rows0, sem_g0).wait()
    pltpu.make_async_copy(src_hbm.at[pl.ds(0, C)], sidx1, sem_s1).wait()
    pltpu.make_async_copy(dst_hbm.at[pl.ds(0, C)], didx1, sem_d1).wait()
    pltpu.make_async_copy(attr_hbm.at[pl.ds(0, C)], attrv1, sem_a1).wait()
    plsc.subcore_barrier()

    pltpu.sync_copy(acc_node.at[pl.ds(r0, RPT)], out_node.at[cid, pl.ds(r0, RPT)])
    pltpu.sync_copy(acc_attr.at[pl.ds(r0, RPT)], out_attr.at[cid, pl.ds(r0, RPT)])
    pltpu.sync_copy(acc_deg.at[pl.ds(r0, RPT)], out_deg.at[cid, pl.ds(r0, RPT)])


def _sc_aggregate(featsS, src, dst, edge_attr, ones_d, zeros_d):
    mesh = plsc.VectorSubcoreMesh(core_axis_name="c", subcore_axis_name="s")
    kfn = pl.kernel(
        _sc_body,
        mesh=mesh,
        compiler_params=pltpu.CompilerParams(use_tc_tiling_on_sc=False),
        out_type=[
            jax.ShapeDtypeStruct((NC, N_PAD, D_HALF), jnp.float32),
            jax.ShapeDtypeStruct((NC, N_PAD, D_EDGE), jnp.float32),
            jax.ShapeDtypeStruct((NC, N_PAD, D_DEG), jnp.float32),
        ],
        scratch_types=[
            pltpu.VMEM((C,), jnp.int32),
            pltpu.VMEM((C,), jnp.int32),
            pltpu.VMEM((C, D_HALF), jnp.float32),
            pltpu.VMEM((C, D_EDGE), jnp.float32),
            pltpu.VMEM((C,), jnp.int32),
            pltpu.VMEM((C,), jnp.int32),
            pltpu.VMEM((C, D_HALF), jnp.float32),
            pltpu.VMEM((C, D_EDGE), jnp.float32),
            pltpu.VMEM((C, D_DEG), jnp.float32),
            pltpu.VMEM_SHARED((N_PAD, D_HALF), jnp.float32),
            pltpu.VMEM_SHARED((N_PAD, D_EDGE), jnp.float32),
            pltpu.VMEM_SHARED((N_PAD, D_DEG), jnp.float32),
            pltpu.SemaphoreType.DMA,
            pltpu.SemaphoreType.DMA,
            pltpu.SemaphoreType.DMA,
            pltpu.SemaphoreType.DMA,
            pltpu.SemaphoreType.DMA,
            pltpu.SemaphoreType.DMA,
            pltpu.SemaphoreType.DMA,
            pltpu.SemaphoreType.DMA,
        ],
    )
    return kfn(featsS, src, dst, edge_attr, ones_d, zeros_d)


def _combine_body(np_ref, ap_ref, dp_ref, feats_ref,
                  wrelt_ref, wedget_ref, wrest_ref,
                  bcomb_ref, bres_ref, gamma_ref, beta_ref, out_ref):
    aggf = jnp.concatenate(
        [np_ref[0, :N_NODES, :], np_ref[1, :N_NODES, :]], axis=1)
    segattr = ap_ref[0, :N_NODES, :]
    deg = dp_ref[1, :N_NODES, 0:1]
    agg = (jnp.dot(aggf, wrelt_ref[...], preferred_element_type=jnp.float32)
           + jnp.dot(segattr, wedget_ref[...], preferred_element_type=jnp.float32)
           + deg * bcomb_ref[...])
    new = jnp.maximum(agg, 0.0)
    res = jnp.maximum(
        jnp.dot(feats_ref[...], wrest_ref[...], preferred_element_type=jnp.float32)
        + bres_ref[...], 0.0)
    new = new + res
    mean = jnp.mean(new, axis=0, keepdims=True)
    var = jnp.mean((new - mean) ** 2, axis=0, keepdims=True)
    out_ref[...] = (new - mean) * lax.rsqrt(var + 1e-5) * gamma_ref[...] + beta_ref[...]


def _combine(node_p, attr_p, deg_p, feats, wrelt, wedget, wrest,
             bcomb, bres, gamma, beta):
    return pl.pallas_call(
        _combine_body,
        out_shape=jax.ShapeDtypeStruct((N_NODES, D_OUT), jnp.float32),
    )(node_p, attr_p, deg_p, feats, wrelt, wedget, wrest, bcomb, bres, gamma, beta)


def kernel(feats, edge_index, edge_attr, W_rel, b_rel, W_edge, b_edge,
           W_res, b_res, gamma, beta):
    src = edge_index[0]
    dst = edge_index[1]
    featsS = jnp.concatenate([feats[:, :D_HALF], feats[:, D_HALF:]], axis=0)
    node_p, attr_p, deg_p = _sc_aggregate(
        featsS, src, dst, edge_attr,
        jnp.ones((C, D_DEG), jnp.float32), jnp.zeros((C, D_DEG), jnp.float32))
    return _combine(
        node_p, attr_p, deg_p, feats,
        W_rel.T, W_edge.T, W_res.T,
        (b_rel + b_edge).reshape(1, D_OUT), b_res.reshape(1, D_OUT),
        gamma.reshape(1, D_OUT), beta.reshape(1, D_OUT))
